# Initial kernel scaffold; baseline (speedup 1.0000x reference)
import jax
import jax.numpy as jnp
from jax.experimental import pallas as pl


def kernel(x, edge_index, W_enc, b_enc, W1, b1, W2, b2, W3, b3):
    raise NotImplementedError("write your pallas kernel here")


# speedup vs baseline: 16.1781x; 16.1781x over previous
import functools

import jax
import jax.numpy as jnp
from jax import lax
from jax.experimental import pallas as pl
from jax.experimental.pallas import tpu as pltpu
from jax.experimental.pallas import tpu_sc as plsc

N = 10000
E = 160000
D = 256
H = 128
NC = 2
NS = 16
CHK = 125
DEG_CH = E // (NC * NS) // CHK
SC_CH = E // NS // CHK
RB = 1000
NRB = N // RB

_mesh = plsc.VectorSubcoreMesh(core_axis_name="c", subcore_axis_name="s")


@functools.partial(
    pl.kernel,
    out_type=jax.ShapeDtypeStruct((NC * N,), jnp.float32),
    mesh=_mesh,
    scratch_types=[
        pltpu.VMEM((DEG_CH, CHK), jnp.int32),
        pltpu.VMEM((128,), jnp.float32),
        pltpu.VMEM((1024,), jnp.float32),
        pltpu.VMEM_SHARED((N,), jnp.float32),
    ],
)
def _deg(dst_hbm, out_hbm, idx_v, ones_v, stage_v, acc_sh):
    c = lax.axis_index("c")
    s = lax.axis_index("s")
    w = c * NS + s

    for k in range(64):
        stage_v[pl.ds(k * 16, 16)] = jnp.zeros((16,), jnp.float32)
    for k in range(8):
        ones_v[pl.ds(k * 16, 16)] = jnp.ones((16,), jnp.float32)

    @pl.when(s < 10)
    def _zero():
        pltpu.sync_copy(stage_v.at[pl.ds(0, 1000)],
                        acc_sh.at[pl.ds(s * 1000, 1000)])

    pltpu.sync_copy(dst_hbm.at[w], idx_v)
    plsc.subcore_barrier()

    def body(j, carry):
        pltpu.sync_copy(ones_v.at[pl.ds(0, CHK)], acc_sh.at[idx_v.at[j]],
                        add=True)
        return carry

    lax.fori_loop(0, DEG_CH, body, 0)
    plsc.subcore_barrier()

    @pl.when(s < 10)
    def _out():
        pltpu.sync_copy(acc_sh.at[pl.ds(s * 1000, 1000)],
                        stage_v.at[pl.ds(0, 1000)])
        pltpu.sync_copy(stage_v.at[pl.ds(0, 1000)],
                        out_hbm.at[pl.ds(c * N + s * 1000, 1000)])


@functools.partial(
    pl.kernel,
    out_type=jax.ShapeDtypeStruct((NC * N, H), jnp.float32),
    mesh=_mesh,
    scratch_types=[
        pltpu.VMEM((SC_CH, CHK), jnp.int32),
        pltpu.VMEM((SC_CH, CHK), jnp.int32),
        pltpu.VMEM((CHK, H), jnp.float32),
        pltpu.VMEM((40, H), jnp.float32),
        pltpu.VMEM_SHARED((N, H), jnp.float32),
        pltpu.SemaphoreType.DMA,
    ],
)
def _scatter(y_hbm, srcoff_hbm, dst_hbm, zrows_hbm, g_hbm,
             idx_s, idx_d, rows_v, stage_v, acc_sh, sem):
    c = lax.axis_index("c")
    s = lax.axis_index("s")

    pltpu.sync_copy(zrows_hbm, stage_v)

    @pl.when(s < 10)
    def _zero():
        def zbody(k, carry):
            pltpu.sync_copy(stage_v, acc_sh.at[pl.ds(s * 1000 + k * 40, 40)])
            return carry

        lax.fori_loop(0, 25, zbody, 0)

    pltpu.sync_copy(srcoff_hbm.at[c, s], idx_s)
    pltpu.sync_copy(dst_hbm.at[s], idx_d)
    plsc.subcore_barrier()

    def body(j, carry):
        pltpu.async_copy(y_hbm.at[idx_s.at[j]], rows_v, sem).wait()
        pltpu.sync_copy(rows_v, acc_sh.at[idx_d.at[j]], add=True)
        return carry

    lax.fori_loop(0, SC_CH, body, 0)
    plsc.subcore_barrier()

    @pl.when(s < 10)
    def _out():
        def obody(k, carry):
            off = pl.multiple_of(s * 1000 + k * 40, 8)
            pltpu.sync_copy(acc_sh.at[pl.ds(off, 40)], stage_v)
            pltpu.sync_copy(stage_v, g_hbm.at[pl.ds(c * N + off, 40)])
            return carry

        lax.fori_loop(0, 25, obody, 0)


def _encode_body(x_ref, w_ref, d0_ref, d1_ref, y_ref):
    deg = d0_ref[...] + d1_ref[...] + 1.0
    dinv = lax.rsqrt(deg)
    y_ref[...] = jnp.dot(x_ref[...], w_ref[...],
                         preferred_element_type=jnp.float32) * dinv


def _encode(x, w_enc, d0, d1):
    return pl.pallas_call(
        _encode_body,
        grid=(2 * NRB,),
        in_specs=[
            pl.BlockSpec((RB, D), lambda p: (p // 2, 0)),
            pl.BlockSpec((D, H), lambda p: (0, p % 2)),
            pl.BlockSpec((RB, 1), lambda p: (p // 2, 0)),
            pl.BlockSpec((RB, 1), lambda p: (p // 2, 0)),
        ],
        out_specs=pl.BlockSpec((RB, H), lambda p: (p // 2 + (p % 2) * NRB, 0)),
        out_shape=jax.ShapeDtypeStruct((NC * N, H), jnp.float32),
    )(x, w_enc, d0, d1)


def _head_body(g0_ref, g1_ref, y0_ref, y1_ref, d0_ref, d1_ref, be_ref,
               w1_ref, b1_ref, w2_ref, b2_ref, w3_ref, b3_ref,
               out_ref, acc_ref):
    i = pl.program_id(0)

    @pl.when(i == 0)
    def _init():
        acc_ref[...] = jnp.zeros_like(acc_ref)

    deg = d0_ref[...] + d1_ref[...] + 1.0
    dinv = lax.rsqrt(deg)
    h0 = jnp.maximum(dinv * (g0_ref[...] + y0_ref[...]) + be_ref[:, 0:H], 0.0)
    h1 = jnp.maximum(dinv * (g1_ref[...] + y1_ref[...]) + be_ref[:, H:D], 0.0)
    acc_ref[0:1, 0:H] += jnp.sum(h0, axis=0, keepdims=True)
    acc_ref[0:1, H:D] += jnp.sum(h1, axis=0, keepdims=True)

    @pl.when(i == pl.num_programs(0) - 1)
    def _mlp():
        emb = acc_ref[...]
        v = jnp.dot(emb, w1_ref[...], preferred_element_type=jnp.float32)
        v = jnp.maximum(v + b1_ref[...], 0.0)
        v = jnp.dot(v, w2_ref[...], preferred_element_type=jnp.float32)
        v = jnp.maximum(v + b2_ref[...], 0.0)
        v = jnp.dot(v, w3_ref[...], preferred_element_type=jnp.float32)
        out_ref[...] = v + b3_ref[...]


def _head(gf, yf, d0, d1, be, w1, b1, w2, b2, w3, b3):
    rowspec = pl.BlockSpec((RB, H), lambda i: (i, 0))
    rowspec1 = pl.BlockSpec((RB, H), lambda i: (i + NRB, 0))
    dspec = pl.BlockSpec((RB, 1), lambda i: (i, 0))

    def full(shape):
        return pl.BlockSpec(shape, lambda i: tuple(0 for _ in shape))

    return pl.pallas_call(
        _head_body,
        grid=(NRB,),
        in_specs=[rowspec, rowspec1, rowspec, rowspec1, dspec, dspec,
                  full((1, D)), full((D, 32)), full((1, 32)),
                  full((32, 32)), full((1, 32)), full((32, 1)), full((1, 1))],
        out_specs=full((1, 1)),
        out_shape=jax.ShapeDtypeStruct((1, 1), jnp.float32),
        scratch_shapes=[pltpu.VMEM((1, D), jnp.float32)],
    )(gf, gf, yf, yf, d0, d1, be, w1, b1, w2, b2, w3, b3)


def kernel(x, edge_index, W_enc, b_enc, W1, b1, W2, b2, W3, b3):
    src = edge_index[0].astype(jnp.int32)
    dst = edge_index[1].astype(jnp.int32)

    dst_a = dst.reshape(NC * NS, DEG_CH, CHK)
    src_b = src.reshape(NS, SC_CH, CHK)
    dst_b = dst.reshape(NS, SC_CH, CHK)
    src_off = jnp.stack([src_b, src_b + N])

    zrows = jnp.zeros((40, H), jnp.float32)

    degf = _deg(dst_a)
    d0 = degf[0:N].reshape(N, 1)
    d1 = degf[N:].reshape(N, 1)

    yf = _encode(x, W_enc, d0, d1)
    gf = _scatter(yf, src_off, dst_b, zrows)

    v = _head(gf, yf, d0, d1, b_enc.reshape(1, D),
              W1, b1.reshape(1, 32), W2, b2.reshape(1, 32),
              W3, b3.reshape(1, 1))
    return v.reshape(1)
